# baseline (speedup 1.0000x reference)
import jax
import jax.numpy as jnp
from jax.experimental import pallas as pl
from jax.experimental.pallas import tpu as pltpu

SIGMA = 1.0
CHUNK = 256


def _lr_kernel(scol_ref, rcol_ref, srow_ref, rrow_ref, loss_ref, pairs_ref):
    b = pl.program_id(0)
    c = pl.program_id(1)

    lane = jax.lax.broadcasted_iota(jnp.int32, (CHUNK, 8), 1)
    bsel = (lane == b).astype(jnp.float32)
    si = jnp.sum(scol_ref[...] * bsel, axis=1, keepdims=True)
    ri = jnp.sum(rcol_ref[...].astype(jnp.float32) * bsel, axis=1,
                 keepdims=True).astype(jnp.int32)
    sj = srow_ref[0]
    rj = rrow_ref[0]

    pos = (ri == 1).astype(jnp.float32)
    neg = (rj == 0).astype(jnp.float32)

    x = SIGMA * (sj - si)
    sp = jnp.maximum(x, 0.0) + jnp.log1p(jnp.exp(-jnp.abs(x)))
    part = jnp.sum(sp * (pos * neg))
    ppairs = jnp.sum(pos) * jnp.sum(neg)

    @pl.when((b == 0) & (c == 0))
    def _init():
        loss_ref[0, 0] = 0.0
        pairs_ref[0, 0] = 0.0

    loss_ref[0, 0] += part
    pairs_ref[0, 0] += ppairs


def kernel(scores, relevances):
    B, N = scores.shape
    rel = relevances.astype(jnp.int32)
    scores_t = scores.T
    rel_t = rel.T

    grid = (B, N // CHUNK)
    loss, pairs = pl.pallas_call(
        _lr_kernel,
        grid=grid,
        in_specs=[
            pl.BlockSpec((CHUNK, 8), lambda b, c: (c, 0)),
            pl.BlockSpec((CHUNK, 8), lambda b, c: (c, 0)),
            pl.BlockSpec((1, 1, N), lambda b, c: (b, 0, 0)),
            pl.BlockSpec((1, 1, N), lambda b, c: (b, 0, 0)),
        ],
        out_specs=[
            pl.BlockSpec(memory_space=pltpu.SMEM),
            pl.BlockSpec(memory_space=pltpu.SMEM),
        ],
        out_shape=[
            jax.ShapeDtypeStruct((1, 1), jnp.float32),
            jax.ShapeDtypeStruct((1, 1), jnp.float32),
        ],
    )(scores_t, rel_t, scores.reshape(B, 1, N), rel.reshape(B, 1, N))

    total = loss[0, 0]
    npairs = pairs[0, 0]
    return jnp.where(npairs > 0, total / npairs, total)


# speedup vs baseline: 1.8803x; 1.8803x over previous
import functools

import jax
import jax.numpy as jnp
from jax import lax
from jax.experimental import pallas as pl
from jax.experimental.pallas import tpu as pltpu
from jax.experimental.pallas import tpu_sc as plsc

SIGMA = 1.0
B = 8
N = 2048
CI = 256
CJ = 512
NJ = N // CJ
L = 16
POS_PAD = 1e30
NEG_PAD = -1e30

def _sc_partition_body(scores_hbm, rel_hbm, pos_hbm, neg_hbm, cnt_hbm,
                       s_v, r_v, pos_v, neg_v, cnt_v):
    wid = lax.axis_index("s") * 2 + lax.axis_index("c")

    @pl.when(wid < B)
    def _():
        b = wid
        pltpu.sync_copy(scores_hbm.at[b], s_v)
        pltpu.sync_copy(rel_hbm.at[b], r_v)

        def initloop(i, carry):
            pos_v[pl.ds(i * L, L)] = jnp.full((L,), POS_PAD, jnp.float32)
            neg_v[pl.ds(i * L, L)] = jnp.full((L,), NEG_PAD, jnp.float32)
            return carry

        lax.fori_loop(0, N // L, initloop, 0)

        lane = lax.iota(jnp.int32, L)

        def chunk(i, carry):
            off_p, off_n = carry
            s = s_v[pl.ds(i * L, L)]
            r = r_v[pl.ds(i * L, L)]
            m = r == 1
            mi = jnp.where(m, 1, 0).astype(jnp.int32)
            incl = plsc.cumsum(mi)
            excl = incl - mi
            npos = jnp.sum(mi)
            pos_idx = lax.broadcast(off_p, (L,)) + excl
            neg_idx = lax.broadcast(off_n, (L,)) + (lane - excl)
            plsc.store_scatter(pos_v, [pos_idx], s, mask=m)
            plsc.store_scatter(neg_v, [neg_idx], s, mask=jnp.logical_not(m))
            return off_p + npos, off_n + (L - npos)

        off_p, _ = lax.fori_loop(0, N // L, chunk,
                                 (jnp.int32(0), jnp.int32(0)))
        cnt_v[...] = lax.broadcast(off_p, (L,))
        pltpu.sync_copy(pos_v, pos_hbm.at[b])
        pltpu.sync_copy(neg_v, neg_hbm.at[b])
        pltpu.sync_copy(cnt_v, cnt_hbm.at[b])


_sc_partition_fn = None


def _sc_partition(scores, rel):
    global _sc_partition_fn
    if _sc_partition_fn is None:
        mesh = plsc.VectorSubcoreMesh(
            core_axis_name="c", subcore_axis_name="s",
            num_cores=2, num_subcores=16)
        _sc_partition_fn = pl.kernel(
            _sc_partition_body,
            compiler_params=pltpu.CompilerParams(needs_layout_passes=False),
            out_type=[
                jax.ShapeDtypeStruct((B, N), jnp.float32),
                jax.ShapeDtypeStruct((B, N), jnp.float32),
                jax.ShapeDtypeStruct((B, L), jnp.int32),
            ],
            mesh=mesh,
            scratch_types=[
                pltpu.VMEM((N,), jnp.float32),
                pltpu.VMEM((N,), jnp.int32),
                pltpu.VMEM((N,), jnp.float32),
                pltpu.VMEM((N,), jnp.float32),
                pltpu.VMEM((L,), jnp.int32),
            ],
        )
    return _sc_partition_fn(scores, rel)


def _softplus(x):
    return jnp.maximum(x, 0.0) + jnp.log1p(jnp.exp(-jnp.abs(x)))


def _pair_kernel(cnt_ref, pos_t_ref, neg_rs_ref, loss_ref, pairs_ref):
    total = jnp.float32(0.0)
    npairs = jnp.float32(0.0)
    for b in range(B):
        p = cnt_ref[b, 0]
        q = N - p
        npairs += (p * q).astype(jnp.float32)
        ni = (p + CI - 1) // CI
        nj = (q + CJ - 1) // CJ

        def iloop(ci, acc, b=b, nj=nj):
            rows = pos_t_ref[pl.ds(ci * CI, CI), b:b + 1]

            def jloop(cj, acc2):
                cols = neg_rs_ref[pl.ds(b * NJ + cj, 1)]
                x = SIGMA * (cols.reshape(1, CJ) - rows)
                return acc2 + jnp.sum(_softplus(x))

            return lax.fori_loop(0, nj, jloop, acc)

        total = lax.fori_loop(0, ni, iloop, total)
    loss_ref[0, 0] = total
    pairs_ref[0, 0] = npairs


def _pairwise_call(cnt, pos_t, neg_rs):
    return pl.pallas_call(
        _pair_kernel,
        in_specs=[
            pl.BlockSpec(memory_space=pltpu.SMEM),
            pl.BlockSpec(memory_space=pltpu.VMEM),
            pl.BlockSpec(memory_space=pltpu.VMEM),
        ],
        out_specs=[
            pl.BlockSpec(memory_space=pltpu.SMEM),
            pl.BlockSpec(memory_space=pltpu.SMEM),
        ],
        out_shape=[
            jax.ShapeDtypeStruct((1, 1), jnp.float32),
            jax.ShapeDtypeStruct((1, 1), jnp.float32),
        ],
    )(cnt, pos_t, neg_rs)


def kernel(scores, relevances):
    rel = relevances.astype(jnp.int32)
    pos, neg, cnt = _sc_partition(scores, rel)
    pos_t = pos.T
    neg_rs = neg.reshape(B * NJ, 1, CJ)
    loss, pairs = _pairwise_call(cnt, pos_t, neg_rs)
    total = loss[0, 0]
    npr = pairs[0, 0]
    return jnp.where(npr > 0, total / npr, total)
